# Initial kernel scaffold; baseline (speedup 1.0000x reference)
import jax
import jax.numpy as jnp
from jax.experimental import pallas as pl


def kernel(positions, node_features, edge_index, W_emb1, b_emb1, W_emb2, b_emb2, W_rbf, b_rbf, W_msg, b_msg, W_upd, b_upd, W_r1, b_r1, W_r2, b_r2):
    raise NotImplementedError("write your pallas kernel here")


# speedup vs baseline: 2.1023x; 2.1023x over previous
import functools

import jax
import jax.numpy as jnp
from jax import lax
from jax.experimental import pallas as pl
from jax.experimental.pallas import tpu as pltpu
from jax.experimental.pallas import tpu_sc as plsc

N = 10000
E = 320000
FEAT = 26
ND = 128
HIDDEN = 256
OUT_DIM = 64
N_LAYERS = 4
N_RBF = 16
CUTOFF = 10.0

NC = 2
NS = 16
NW = NC * NS
EC = 128
NCHUNK = E // EC
KMAX = -(-NCHUNK // NW)
N_PAD = 10240
ROWS_PER_TILE = N_PAD // NS

@functools.cache
def _sc_mesh():
    return plsc.VectorSubcoreMesh(core_axis_name="c", subcore_axis_name="s")


def _silu(x):
    return x / (1.0 + jnp.exp(-x))


def _emb_body(nf, w1, b1, w2, b2, wm, bm, h_out, hm_out):
    h = _silu(nf[...] @ w1[...] + b1[...]) @ w2[...] + b2[...]
    h_out[...] = h
    hm_out[...] = h @ wm[...] + bm[...]


def _tc_embed(nf, w1, b1, w2, b2, wm, bm):
    return pl.pallas_call(
        _emb_body,
        out_shape=(
            jax.ShapeDtypeStruct((N, ND), jnp.float32),
            jax.ShapeDtypeStruct((N, ND), jnp.float32),
        ),
    )(nf, w1, b1, w2, b2, wm, bm)


def _gate_body(src4, dst4, w, b, gate_out):
    rel = dst4[...] - src4[...]
    d = jnp.sqrt(jnp.sum(rel * rel, axis=1, keepdims=True) + 1e-12)
    mu = (CUTOFF / (N_RBF - 1)) * lax.broadcasted_iota(
        jnp.int32, (1, N_RBF), 1).astype(jnp.float32)
    sigma = CUTOFF / N_RBF
    basis = jnp.exp(-0.5 * ((d - mu) / sigma) ** 2)
    env = 0.5 * (jnp.cos(jnp.pi * jnp.clip(d / CUTOFF, 0.0, 1.0)) + 1.0)
    feat = basis * env
    gate_out[...] = _silu(feat @ w[...] + b[...])


_GATE_BLK = 2560


def _tc_gates(src4, dst4, w, b):
    nblk = E // _GATE_BLK
    return pl.pallas_call(
        _gate_body,
        grid=(nblk,),
        in_specs=[
            pl.BlockSpec((_GATE_BLK, 8), lambda i: (i, 0)),
            pl.BlockSpec((_GATE_BLK, 8), lambda i: (i, 0)),
            pl.BlockSpec((N_RBF, ND), lambda i: (0, 0)),
            pl.BlockSpec((1, ND), lambda i: (0, 0)),
        ],
        out_specs=pl.BlockSpec((_GATE_BLK, ND), lambda i: (i, 0)),
        out_shape=jax.ShapeDtypeStruct((E, ND), jnp.float32),
    )(src4, dst4, w, b)


def _upd_body(h, agg2, wu_h, wu_a, bu, wm, bm, h_out, hm_out):
    agg = agg2[0, :N] + agg2[1, :N]
    u = _silu(h[...] @ wu_h[...] + agg @ wu_a[...] + bu[...])
    hn = h[...] + u
    h_out[...] = hn
    hm_out[...] = hn @ wm[...] + bm[...]


def _tc_update(h, agg2, wu_h, wu_a, bu, wm, bm):
    return pl.pallas_call(
        _upd_body,
        out_shape=(
            jax.ShapeDtypeStruct((N, ND), jnp.float32),
            jax.ShapeDtypeStruct((N, ND), jnp.float32),
        ),
    )(h, agg2, wu_h, wu_a, bu, wm, bm)


def _final_body(h, agg2, wu_h, wu_a, bu, wr1, br1, wr2, br2, out):
    agg = agg2[0, :N] + agg2[1, :N]
    u = _silu(h[...] @ wu_h[...] + agg @ wu_a[...] + bu[...])
    hn = h[...] + u
    g = jnp.mean(hn, axis=0, keepdims=True)
    out[...] = _silu(g @ wr1[...] + br1[...]) @ wr2[...] + br2[...]


def _tc_final(h, agg2, wu_h, wu_a, bu, wr1, br1, wr2, br2):
    return pl.pallas_call(
        _final_body,
        out_shape=jax.ShapeDtypeStruct((1, OUT_DIM), jnp.float32),
    )(h, agg2, wu_h, wu_a, bu, wr1, br1, wr2, br2)


def _sc_wid():
    return lax.axis_index("s") * NC + lax.axis_index("c")


def _posgather_kernel(pos8, src, dst, src8_out, dst8_out, sidx, didx,
                      srows, drows, sem):
    wid = _sc_wid()

    def body(k, _):
        chunk = wid + k * NW

        @pl.when(chunk < NCHUNK)
        def _():
            base = chunk * EC
            pltpu.sync_copy(src.at[pl.ds(base, EC)], sidx)
            pltpu.sync_copy(dst.at[pl.ds(base, EC)], didx)
            pltpu.async_copy(pos8.at[sidx], srows, sem).wait()
            pltpu.async_copy(pos8.at[didx], drows, sem).wait()
            pltpu.sync_copy(srows, src8_out.at[pl.ds(base, EC)])
            pltpu.sync_copy(drows, dst8_out.at[pl.ds(base, EC)])
        return 0

    lax.fori_loop(0, KMAX, body, 0)


@functools.cache
def _sc_posgather_fn():
    return pl.kernel(
        _posgather_kernel,
        out_type=(
            jax.ShapeDtypeStruct((E, 8), jnp.float32),
            jax.ShapeDtypeStruct((E, 8), jnp.float32),
        ),
        mesh=_sc_mesh(),
        scratch_types=[
            pltpu.VMEM((EC,), jnp.int32),
            pltpu.VMEM((EC,), jnp.int32),
            pltpu.VMEM((EC, 8), jnp.float32),
            pltpu.VMEM((EC, 8), jnp.float32),
            pltpu.SemaphoreType.DMA,
        ],
        compiler_params=pltpu.CompilerParams(use_tc_tiling_on_sc=False),
    )


def _sc_posgather(pos8, src, dst):
    return _sc_posgather_fn()(pos8, src, dst)


def _edge_kernel(hm, gate, src, dst, zeros, agg_out, sidx, didx, rows,
                 gatev, agg_sh, sem):
    cid = lax.axis_index("c")
    sid = lax.axis_index("s")
    wid = sid * NC + cid

    pltpu.sync_copy(zeros, agg_sh.at[pl.ds(sid * ROWS_PER_TILE, ROWS_PER_TILE)])
    plsc.subcore_barrier()

    def body(k, _):
        chunk = wid + k * NW

        @pl.when(chunk < NCHUNK)
        def _():
            base = chunk * EC
            pltpu.sync_copy(src.at[pl.ds(base, EC)], sidx)
            pltpu.sync_copy(gate.at[pl.ds(base, EC), :], gatev)
            pltpu.async_copy(hm.at[sidx], rows, sem).wait()

            def mul_row(j, _c):
                def mul_vec(i, _c2):
                    sl = pl.ds(i * 16, 16)
                    rows[j, sl] = rows[j, sl] * gatev[j, sl]
                    return 0
                lax.fori_loop(0, ND // 16, mul_vec, 0)
                return 0
            lax.fori_loop(0, EC, mul_row, 0)

            pltpu.sync_copy(dst.at[pl.ds(base, EC)], didx)
            pltpu.sync_copy(rows, agg_sh.at[didx], add=True)
        return 0

    lax.fori_loop(0, KMAX, body, 0)
    plsc.subcore_barrier()
    sl = pl.ds(sid * ROWS_PER_TILE, ROWS_PER_TILE)
    pltpu.sync_copy(agg_sh.at[sl],
                    agg_out.at[pl.ds(cid * N_PAD + sid * ROWS_PER_TILE,
                                     ROWS_PER_TILE)])


@functools.cache
def _sc_edge_fn():
    return pl.kernel(
        _edge_kernel,
        out_type=jax.ShapeDtypeStruct((NC * N_PAD, ND), jnp.float32),
        mesh=_sc_mesh(),
        scratch_types=[
            pltpu.VMEM((EC,), jnp.int32),
            pltpu.VMEM((EC,), jnp.int32),
            pltpu.VMEM((EC, ND), jnp.float32),
            pltpu.VMEM((EC, ND), jnp.float32),
            pltpu.VMEM_SHARED((N_PAD, ND), jnp.float32),
            pltpu.SemaphoreType.DMA,
        ],
    )


def _sc_edge(hm, gate, src, dst, zeros):
    return _sc_edge_fn()(hm, gate, src, dst, zeros)


def kernel(positions, node_features, edge_index, W_emb1, b_emb1, W_emb2,
           b_emb2, W_rbf, b_rbf, W_msg, b_msg, W_upd, b_upd, W_r1, b_r1,
           W_r2, b_r2):
    src = edge_index[0]
    dst = edge_index[1]
    pos8 = jnp.pad(positions, ((0, 0), (0, 5)))
    zeros = jnp.zeros((ROWS_PER_TILE, ND), jnp.float32)

    h, hm = _tc_embed(node_features, W_emb1, b_emb1.reshape(1, ND), W_emb2,
                      b_emb2.reshape(1, ND), W_msg[0], b_msg[0].reshape(1, ND))

    src8, dst8 = _sc_posgather(pos8, src, dst)

    for l in range(N_LAYERS):
        gate = _tc_gates(src8, dst8, W_rbf[l], b_rbf[l].reshape(1, ND))
        agg2 = _sc_edge(hm, gate, src, dst, zeros).reshape(NC, N_PAD, ND)
        wu_h = W_upd[l][:ND]
        wu_a = W_upd[l][ND:]
        if l < N_LAYERS - 1:
            h, hm = _tc_update(h, agg2, wu_h, wu_a, b_upd[l].reshape(1, ND),
                               W_msg[l + 1], b_msg[l + 1].reshape(1, ND))
        else:
            out = _tc_final(h, agg2, wu_h, wu_a, b_upd[l].reshape(1, ND),
                            W_r1, b_r1.reshape(1, HIDDEN), W_r2,
                            b_r2.reshape(1, OUT_DIM))
    return out


# speedup vs baseline: 2.1455x; 1.0206x over previous
import functools

import jax
import jax.numpy as jnp
from jax import lax
from jax.experimental import pallas as pl
from jax.experimental.pallas import tpu as pltpu
from jax.experimental.pallas import tpu_sc as plsc

N = 10000
E = 320000
FEAT = 26
ND = 128
HIDDEN = 256
OUT_DIM = 64
N_LAYERS = 4
N_RBF = 16
CUTOFF = 10.0

NC = 2
NS = 16
NW = NC * NS
EC = 80
CHW = 128
E_PAD = NW * CHW * EC
NCHUNK = E_PAD // EC
KMAX = NCHUNK // NW
N_PAD = 10240
ROWS_PER_TILE = N_PAD // NS

@functools.cache
def _sc_mesh():
    return plsc.VectorSubcoreMesh(core_axis_name="c", subcore_axis_name="s")


def _silu(x):
    return x / (1.0 + jnp.exp(-x))


def _emb_body(nf, w1, b1, w2, b2, wm, bm, h_out, hm_out):
    h = _silu(nf[...] @ w1[...] + b1[...]) @ w2[...] + b2[...]
    h_out[...] = h
    hm_out[...] = h @ wm[...] + bm[...]


def _tc_embed(nf, w1, b1, w2, b2, wm, bm):
    return pl.pallas_call(
        _emb_body,
        out_shape=(
            jax.ShapeDtypeStruct((N, ND), jnp.float32),
            jax.ShapeDtypeStruct((N, ND), jnp.float32),
        ),
    )(nf, w1, b1, w2, b2, wm, bm)


def _gate_body(src4, dst4, w, b, gate_out):
    rel = dst4[...] - src4[...]
    d = jnp.sqrt(jnp.sum(rel * rel, axis=1, keepdims=True) + 1e-12)
    mu = (CUTOFF / (N_RBF - 1)) * lax.broadcasted_iota(
        jnp.int32, (1, N_RBF), 1).astype(jnp.float32)
    sigma = CUTOFF / N_RBF
    basis = jnp.exp(-0.5 * ((d - mu) / sigma) ** 2)
    env = 0.5 * (jnp.cos(jnp.pi * jnp.clip(d / CUTOFF, 0.0, 1.0)) + 1.0)
    feat = basis * env
    gate_out[...] = _silu(feat @ w[...] + b[...])


_GATE_BLK = 2560


def _tc_gates(src4, dst4, w, b):
    nblk = E_PAD // _GATE_BLK
    return pl.pallas_call(
        _gate_body,
        grid=(nblk,),
        in_specs=[
            pl.BlockSpec((_GATE_BLK, 8), lambda i: (i, 0)),
            pl.BlockSpec((_GATE_BLK, 8), lambda i: (i, 0)),
            pl.BlockSpec((N_RBF, ND), lambda i: (0, 0)),
            pl.BlockSpec((1, ND), lambda i: (0, 0)),
        ],
        out_specs=pl.BlockSpec((_GATE_BLK, ND), lambda i: (i, 0)),
        out_shape=jax.ShapeDtypeStruct((E_PAD, ND), jnp.float32),
    )(src4, dst4, w, b)


def _upd_body(h, agg2, wu_h, wu_a, bu, wm, bm, h_out, hm_out):
    agg = agg2[0, :N] + agg2[1, :N]
    u = _silu(h[...] @ wu_h[...] + agg @ wu_a[...] + bu[...])
    hn = h[...] + u
    h_out[...] = hn
    hm_out[...] = hn @ wm[...] + bm[...]


def _tc_update(h, agg2, wu_h, wu_a, bu, wm, bm):
    return pl.pallas_call(
        _upd_body,
        out_shape=(
            jax.ShapeDtypeStruct((N, ND), jnp.float32),
            jax.ShapeDtypeStruct((N, ND), jnp.float32),
        ),
    )(h, agg2, wu_h, wu_a, bu, wm, bm)


def _final_body(h, agg2, wu_h, wu_a, bu, wr1, br1, wr2, br2, out):
    agg = agg2[0, :N] + agg2[1, :N]
    u = _silu(h[...] @ wu_h[...] + agg @ wu_a[...] + bu[...])
    hn = h[...] + u
    g = jnp.mean(hn, axis=0, keepdims=True)
    out[...] = _silu(g @ wr1[...] + br1[...]) @ wr2[...] + br2[...]


def _tc_final(h, agg2, wu_h, wu_a, bu, wr1, br1, wr2, br2):
    return pl.pallas_call(
        _final_body,
        out_shape=jax.ShapeDtypeStruct((1, OUT_DIM), jnp.float32),
    )(h, agg2, wu_h, wu_a, bu, wr1, br1, wr2, br2)


def _sc_wid():
    return lax.axis_index("s") * NC + lax.axis_index("c")


def _posgather_kernel(pos8, src, dst, src8_out, dst8_out, sidx, didx,
                      srows, drows, sem):
    wid = _sc_wid()

    def body(k, _):
        base = (wid * CHW + k) * EC
        pltpu.sync_copy(src.at[pl.ds(base, EC)], sidx)
        pltpu.sync_copy(dst.at[pl.ds(base, EC)], didx)
        pltpu.async_copy(pos8.at[sidx], srows, sem).wait()
        pltpu.async_copy(pos8.at[didx], drows, sem).wait()
        pltpu.sync_copy(srows, src8_out.at[pl.ds(base, EC)])
        pltpu.sync_copy(drows, dst8_out.at[pl.ds(base, EC)])
        return 0

    lax.fori_loop(0, KMAX, body, 0)


@functools.cache
def _sc_posgather_fn():
    return pl.kernel(
        _posgather_kernel,
        out_type=(
            jax.ShapeDtypeStruct((E_PAD, 8), jnp.float32),
            jax.ShapeDtypeStruct((E_PAD, 8), jnp.float32),
        ),
        mesh=_sc_mesh(),
        scratch_types=[
            pltpu.VMEM((EC,), jnp.int32),
            pltpu.VMEM((EC,), jnp.int32),
            pltpu.VMEM((EC, 8), jnp.float32),
            pltpu.VMEM((EC, 8), jnp.float32),
            pltpu.SemaphoreType.DMA,
        ],
        compiler_params=pltpu.CompilerParams(use_tc_tiling_on_sc=False),
    )


def _sc_posgather(pos8, src, dst):
    return _sc_posgather_fn()(pos8, src, dst)


def _edge_kernel(hm, gate, src, dst, zeros, agg_out, sidx, didx,
                 rows0, rows1, gate0, gate1, agg_sh,
                 semi, semg0, semg1, semt0, semt1):
    cid = lax.axis_index("c")
    sid = lax.axis_index("s")
    wid = sid * NC + cid
    c0 = wid * CHW

    pltpu.sync_copy(zeros, agg_sh.at[pl.ds(sid * ROWS_PER_TILE, ROWS_PER_TILE)])
    plsc.subcore_barrier()

    def idx_fetch(k, slot):
        base = (c0 + k) * EC
        pltpu.async_copy(src.at[pl.ds(base, EC)], sidx.at[slot], semi)
        pltpu.async_copy(dst.at[pl.ds(base, EC)], didx.at[slot], semi)

    def idx_wait():
        pltpu.make_async_copy(src.at[pl.ds(0, EC)], sidx.at[0], semi).wait()
        pltpu.make_async_copy(dst.at[pl.ds(0, EC)], didx.at[0], semi).wait()

    def start(k, slot, rows, gatev, semg, semt):
        base = (c0 + k) * EC
        pltpu.async_copy(gate.at[pl.ds(base, EC), :], gatev, semt)
        pltpu.async_copy(hm.at[sidx.at[slot]], rows, semg)

    def finish(slot, rows, gatev, semg, semt):
        pltpu.make_async_copy(gate.at[pl.ds(0, EC), :], gatev, semt).wait()
        pltpu.make_async_copy(hm.at[sidx.at[0]], rows, semg).wait()

        @plsc.parallel_loop(0, EC, 1, unroll=4)
        def _mul(j):
            for i in range(ND // 16):
                sl = pl.ds(i * 16, 16)
                rows[j, sl] = rows[j, sl] * gatev[j, sl]

        pltpu.sync_copy(rows, agg_sh.at[didx.at[slot]], add=True)

    idx_fetch(0, 0)
    idx_fetch(1, 1)
    idx_wait()
    idx_wait()
    start(0, 0, rows0, gate0, semg0, semt0)

    def body(kp, _):
        k0 = kp * 2
        s0 = lax.rem(k0, 4)
        s1 = lax.rem(k0 + 1, 4)
        s2 = lax.rem(k0 + 2, 4)
        s3 = lax.rem(k0 + 3, 4)
        more = kp < CHW // 2 - 1

        @pl.when(more)
        def _():
            idx_fetch(k0 + 2, s2)
            idx_fetch(k0 + 3, s3)
        start(k0 + 1, s1, rows1, gate1, semg1, semt1)
        finish(s0, rows0, gate0, semg0, semt0)

        @pl.when(more)
        def _():
            idx_wait()
            idx_wait()
            start(k0 + 2, s2, rows0, gate0, semg0, semt0)
        finish(s1, rows1, gate1, semg1, semt1)
        return 0

    lax.fori_loop(0, CHW // 2, body, 0)

    plsc.subcore_barrier()
    sl = pl.ds(sid * ROWS_PER_TILE, ROWS_PER_TILE)
    pltpu.sync_copy(agg_sh.at[sl],
                    agg_out.at[pl.ds(cid * N_PAD + sid * ROWS_PER_TILE,
                                     ROWS_PER_TILE)])


@functools.cache
def _sc_edge_fn():
    return pl.kernel(
        _edge_kernel,
        out_type=jax.ShapeDtypeStruct((NC * N_PAD, ND), jnp.float32),
        mesh=_sc_mesh(),
        scratch_types=[
            pltpu.VMEM((4, EC), jnp.int32),
            pltpu.VMEM((4, EC), jnp.int32),
            pltpu.VMEM((EC, ND), jnp.float32),
            pltpu.VMEM((EC, ND), jnp.float32),
            pltpu.VMEM((EC, ND), jnp.float32),
            pltpu.VMEM((EC, ND), jnp.float32),
            pltpu.VMEM_SHARED((N_PAD, ND), jnp.float32),
            pltpu.SemaphoreType.DMA,
            pltpu.SemaphoreType.DMA,
            pltpu.SemaphoreType.DMA,
            pltpu.SemaphoreType.DMA,
            pltpu.SemaphoreType.DMA,
        ],
    )


def _sc_edge(hm, gate, src, dst, zeros):
    return _sc_edge_fn()(hm, gate, src, dst, zeros)


def kernel(positions, node_features, edge_index, W_emb1, b_emb1, W_emb2,
           b_emb2, W_rbf, b_rbf, W_msg, b_msg, W_upd, b_upd, W_r1, b_r1,
           W_r2, b_r2):
    npad = E_PAD - E
    pad_iota = jnp.arange(npad, dtype=jnp.int32)
    src = jnp.concatenate([edge_index[0], pad_iota % N])
    dst = jnp.concatenate([edge_index[1], N + pad_iota % (N_PAD - N)])
    pos8 = jnp.pad(positions, ((0, 0), (0, 5)))
    zeros = jnp.zeros((ROWS_PER_TILE, ND), jnp.float32)

    h, hm = _tc_embed(node_features, W_emb1, b_emb1.reshape(1, ND), W_emb2,
                      b_emb2.reshape(1, ND), W_msg[0], b_msg[0].reshape(1, ND))

    src8, dst8 = _sc_posgather(pos8, src, dst)
    gates = [_tc_gates(src8, dst8, W_rbf[l], b_rbf[l].reshape(1, ND))
             for l in range(N_LAYERS)]

    for l in range(N_LAYERS):
        agg2 = _sc_edge(hm, gates[l], src, dst, zeros).reshape(NC, N_PAD, ND)
        wu_h = W_upd[l][:ND]
        wu_a = W_upd[l][ND:]
        if l < N_LAYERS - 1:
            h, hm = _tc_update(h, agg2, wu_h, wu_a, b_upd[l].reshape(1, ND),
                               W_msg[l + 1], b_msg[l + 1].reshape(1, ND))
        else:
            out = _tc_final(h, agg2, wu_h, wu_a, b_upd[l].reshape(1, ND),
                            W_r1, b_r1.reshape(1, HIDDEN), W_r2,
                            b_r2.reshape(1, OUT_DIM))
    return out
